# Initial kernel scaffold; baseline (speedup 1.0000x reference)
import jax
import jax.numpy as jnp
from jax.experimental import pallas as pl


def kernel(edge_index, node_sh):
    raise NotImplementedError("write your pallas kernel here")


# speedup vs baseline: 6.3528x; 6.3528x over previous
import dataclasses
import functools

import jax
import jax.numpy as jnp
from jax import lax
from jax.experimental import pallas as pl
from jax.experimental.pallas import tpu as pltpu
from jax.experimental.pallas import tpu_sc as plsc

_NC = 2
_NS = 16
_NW = _NC * _NS
_L = 16
_G = 128
_GPC = 8
_W = _G * _GPC


def _take16(c, idx):
    dnums = lax.GatherDimensionNumbers(
        offset_dims=(), collapsed_slice_dims=(0,), start_index_map=(0,))
    return lax.gather(c, idx[:, None], dnums, slice_sizes=(1,),
                      mode=lax.GatherScatterMode.PROMISE_IN_BOUNDS)


def _sc_kernel_body(node_hbm, row_hbm, col_hbm, out_hbm,
                    idx_a, idx_b, arows, brows, out_v, sem):
    n_chunks = row_hbm.shape[0] // (_NW * _GPC)
    wid = lax.axis_index("s") * _NC + lax.axis_index("c")
    base_g = wid * n_chunks * _GPC

    lane = lax.iota(jnp.int32, _L)
    q = lane & 3
    q2 = (q + 3) & 3
    idx_hi = q * q + 2 * q
    idx_lo = q2 * q2 + 2 * q2
    seg_mask = q >= 1
    mask4 = lane < 4

    @pl.loop(0, n_chunks)
    def _chunk(k):
        g0 = base_g + k * _GPC
        pltpu.sync_copy(row_hbm.at[pl.ds(g0, _GPC)], idx_a)
        pltpu.sync_copy(col_hbm.at[pl.ds(g0, _GPC)], idx_b)
        copies = []
        for j in range(_GPC):
            copies.append(pltpu.async_copy(
                node_hbm.at[idx_a.at[j]],
                arows.at[pl.ds(j * _G, _G)], sem))
            copies.append(pltpu.async_copy(
                node_hbm.at[idx_b.at[j]],
                brows.at[pl.ds(j * _G, _G)], sem))
        for c in copies:
            c.wait()

        @pl.loop(0, _W)
        def _edge(e):
            t = arows[e, :] * brows[e, :]
            c = plsc.cumsum(t)
            hi = _take16(c, idx_hi)
            lo = _take16(c, idx_lo)
            d = hi - jnp.where(seg_mask, lo, jnp.float32(0.0))
            erow = jnp.full((_L,), e, dtype=jnp.int32)
            plsc.store_scatter(out_v, [erow, q], d, mask=mask4)

        pltpu.sync_copy(out_v, out_hbm.at[pl.ds(g0 * _G, _W)])


@functools.partial(jax.jit, static_argnames=("e_pad",))
def _sc_call(node_sh, row2, col2, e_pad):
    mesh = plsc.VectorSubcoreMesh(core_axis_name="c", subcore_axis_name="s")
    cp = pltpu.CompilerParams()
    if "needs_layout_passes" in pltpu.CompilerParams.__dataclass_fields__:
        cp = dataclasses.replace(cp, needs_layout_passes=False)
    if "use_tc_tiling_on_sc" in pltpu.CompilerParams.__dataclass_fields__:
        cp = dataclasses.replace(cp, use_tc_tiling_on_sc=False)
    kfn = pl.kernel(
        _sc_kernel_body,
        out_type=jax.ShapeDtypeStruct((e_pad, 4), jnp.float32),
        mesh=mesh,
        scratch_types=[
            pltpu.VMEM((_GPC, _G), jnp.int32),
            pltpu.VMEM((_GPC, _G), jnp.int32),
            pltpu.VMEM((_W, _L), jnp.float32),
            pltpu.VMEM((_W, _L), jnp.float32),
            pltpu.VMEM((_W, 4), jnp.float32),
            pltpu.SemaphoreType.DMA,
        ],
        compiler_params=cp,
    )
    return kfn(node_sh, row2, col2)


def kernel(edge_index, node_sh):
    e = edge_index.shape[1]
    per_tile = -(-e // (_NW * _W)) * _W
    e_pad = per_tile * _NW
    ei = edge_index.astype(jnp.int32)
    row = jnp.pad(ei[0], (0, e_pad - e))
    col = jnp.pad(ei[1], (0, e_pad - e))
    out = _sc_call(node_sh, row.reshape(-1, _G), col.reshape(-1, _G), e_pad)
    return out[:e]
